# Initial kernel scaffold; baseline (speedup 1.0000x reference)
import jax
import jax.numpy as jnp
from jax.experimental import pallas as pl


def kernel(user, movie, user_table, movie_table, W1, b1, W2, b2):
    raise NotImplementedError("write your pallas kernel here")


# speedup vs baseline: 1.1656x; 1.1656x over previous
import functools

import jax
import jax.numpy as jnp
from jax import lax
from jax.experimental import pallas as pl
from jax.experimental.pallas import tpu as pltpu
from jax.experimental.pallas import tpu_sc as plsc

_NC = 2
_NS = 16
_NW = _NC * _NS

_BATCH = 16384
_EMBED = 64
_B_PER_W = _BATCH // _NW


def _sc_gather_body(user_idx_hbm, movie_idx_hbm, user_table_hbm,
                    movie_table_hbm, out_u_hbm, out_m_hbm,
                    uidx_v, midx_v, urows_v, mrows_v, sem_u, sem_m):
    wid = lax.axis_index("s") * _NC + lax.axis_index("c")
    base = wid * _B_PER_W
    pltpu.sync_copy(user_idx_hbm.at[pl.ds(base, _B_PER_W)], uidx_v)
    pltpu.sync_copy(movie_idx_hbm.at[pl.ds(base, _B_PER_W)], midx_v)
    cp_u = pltpu.async_copy(user_table_hbm.at[uidx_v], urows_v, sem_u)
    cp_m = pltpu.async_copy(movie_table_hbm.at[midx_v], mrows_v, sem_m)
    cp_u.wait()
    cp_m.wait()
    pltpu.sync_copy(urows_v, out_u_hbm.at[pl.ds(base, _B_PER_W)])
    pltpu.sync_copy(mrows_v, out_m_hbm.at[pl.ds(base, _B_PER_W)])


def _sc_gather(user_idx, movie_idx, user_table, movie_table):
    mesh = plsc.VectorSubcoreMesh(core_axis_name="c", subcore_axis_name="s",
                                  num_cores=_NC, num_subcores=_NS)
    return pl.kernel(
        _sc_gather_body,
        out_type=[
            jax.ShapeDtypeStruct((_BATCH, _EMBED), jnp.float32),
            jax.ShapeDtypeStruct((_BATCH, _EMBED), jnp.float32),
        ],
        mesh=mesh,
        scratch_types=[
            pltpu.VMEM((_B_PER_W,), jnp.int32),
            pltpu.VMEM((_B_PER_W,), jnp.int32),
            pltpu.VMEM((_B_PER_W, _EMBED), jnp.float32),
            pltpu.VMEM((_B_PER_W, _EMBED), jnp.float32),
            pltpu.SemaphoreType.DMA,
            pltpu.SemaphoreType.DMA,
        ],
        compiler_params=pltpu.CompilerParams(use_tc_tiling_on_sc=False),
    )(user_idx, movie_idx, user_table, movie_table)


def _mlp_body(u_ref, m_ref, w1_ref, b1_ref, w2_ref, b2_ref, out_ref):
    u = u_ref[...]
    m = m_ref[...]
    w1 = w1_ref[...]
    dn = (((1,), (1,)), ((), ()))
    h = lax.dot_general(u, w1[:, :_EMBED], dn,
                        preferred_element_type=jnp.float32)
    h += lax.dot_general(m, w1[:, _EMBED:], dn,
                         preferred_element_type=jnp.float32)
    h = jnp.maximum(h + b1_ref[...], 0.0)
    y = jnp.sum(h * w2_ref[...], axis=1, keepdims=True) + b2_ref[...]
    out_ref[...] = y


def _mlp(u_rows, m_rows, W1, b1, W2, b2):
    BR = 2048
    grid = (_BATCH // BR,)
    return pl.pallas_call(
        _mlp_body,
        grid=grid,
        in_specs=[
            pl.BlockSpec((BR, _EMBED), lambda i: (i, 0)),
            pl.BlockSpec((BR, _EMBED), lambda i: (i, 0)),
            pl.BlockSpec((128, 2 * _EMBED), lambda i: (0, 0)),
            pl.BlockSpec((1, 128), lambda i: (0, 0)),
            pl.BlockSpec((1, 128), lambda i: (0, 0)),
            pl.BlockSpec((1, 1), lambda i: (0, 0)),
        ],
        out_specs=pl.BlockSpec((BR, 1), lambda i: (i, 0)),
        out_shape=jax.ShapeDtypeStruct((_BATCH, 1), jnp.float32),
    )(u_rows, m_rows, W1, b1.reshape(1, 128), W2, b2.reshape(1, 1))


@jax.jit
def kernel(user, movie, user_table, movie_table, W1, b1, W2, b2):
    u_rows, m_rows = _sc_gather(user, movie, user_table, movie_table)
    y = _mlp(u_rows, m_rows, W1, b1, W2, b2)
    return y.reshape(_BATCH)


# speedup vs baseline: 1.3929x; 1.1950x over previous
import jax
import jax.numpy as jnp
from jax import lax
from jax.experimental import pallas as pl
from jax.experimental.pallas import tpu as pltpu
from jax.experimental.pallas import tpu_sc as plsc

_NC = 2
_NS = 16
_NW = _NC * _NS

_BATCH = 16384
_EMBED = 64
_ROW = 2 * _EMBED
_B_PER_W = _BATCH // _NW
_CHUNK = 256
_N_CHUNKS = _B_PER_W // _CHUNK


def _sc_gather_body(user_idx_hbm, movie_idx_hbm, table_hbm,
                    out_u_hbm, out_m_hbm,
                    uidx_v, midx_v, ubuf_v, mbuf_v, sem_u, sem_m):
    wid = lax.axis_index("s") * _NC + lax.axis_index("c")
    base = wid * _B_PER_W
    pltpu.sync_copy(user_idx_hbm.at[pl.ds(base, _B_PER_W)], uidx_v)
    pltpu.sync_copy(movie_idx_hbm.at[pl.ds(base, _B_PER_W)], midx_v)
    for ch in range(_N_CHUNKS):
        off = ch * _CHUNK
        cp_u = pltpu.async_copy(table_hbm.at[uidx_v.at[pl.ds(off, _CHUNK)]],
                                ubuf_v, sem_u)
        cp_m = pltpu.async_copy(table_hbm.at[midx_v.at[pl.ds(off, _CHUNK)]],
                                mbuf_v, sem_m)
        cp_u.wait()
        cp_m.wait()
        pltpu.sync_copy(ubuf_v, out_u_hbm.at[pl.ds(base + off, _CHUNK)])
        pltpu.sync_copy(mbuf_v, out_m_hbm.at[pl.ds(base + off, _CHUNK)])


def _sc_gather(user_idx, movie_idx, table):
    mesh = plsc.VectorSubcoreMesh(core_axis_name="c", subcore_axis_name="s",
                                  num_cores=_NC, num_subcores=_NS)
    return pl.kernel(
        _sc_gather_body,
        out_type=[
            jax.ShapeDtypeStruct((_BATCH, _ROW), jnp.float32),
            jax.ShapeDtypeStruct((_BATCH, _ROW), jnp.float32),
        ],
        mesh=mesh,
        scratch_types=[
            pltpu.VMEM((_B_PER_W,), jnp.int32),
            pltpu.VMEM((_B_PER_W,), jnp.int32),
            pltpu.VMEM((_CHUNK, _ROW), jnp.float32),
            pltpu.VMEM((_CHUNK, _ROW), jnp.float32),
            pltpu.SemaphoreType.DMA,
            pltpu.SemaphoreType.DMA,
        ],
    )(user_idx, movie_idx, table)


def _mlp_body(u_ref, m_ref, w1_ref, b1_ref, w2_ref, b2_ref, out_ref):
    u = u_ref[...]
    m = m_ref[...]
    w1 = w1_ref[...]
    dn = (((1,), (1,)), ((), ()))
    h = lax.dot_general(u[:, :_EMBED], w1[:, :_EMBED], dn,
                        preferred_element_type=jnp.float32)
    h += lax.dot_general(m[:, _EMBED:], w1[:, _EMBED:], dn,
                         preferred_element_type=jnp.float32)
    h = jnp.maximum(h + b1_ref[...], 0.0)
    y = jnp.sum(h * w2_ref[...], axis=1, keepdims=True) + b2_ref[...]
    out_ref[...] = y


def _mlp(u_rows, m_rows, W1, b1, W2, b2):
    BR = 2048
    grid = (_BATCH // BR,)
    return pl.pallas_call(
        _mlp_body,
        grid=grid,
        in_specs=[
            pl.BlockSpec((BR, _ROW), lambda i: (i, 0)),
            pl.BlockSpec((BR, _ROW), lambda i: (i, 0)),
            pl.BlockSpec((128, _ROW), lambda i: (0, 0)),
            pl.BlockSpec((1, 128), lambda i: (0, 0)),
            pl.BlockSpec((1, 128), lambda i: (0, 0)),
            pl.BlockSpec((1, 1), lambda i: (0, 0)),
        ],
        out_specs=pl.BlockSpec((BR, 1), lambda i: (i, 0)),
        out_shape=jax.ShapeDtypeStruct((_BATCH, 1), jnp.float32),
    )(u_rows, m_rows, W1, b1.reshape(1, 128), W2, b2.reshape(1, 1))


@jax.jit
def kernel(user, movie, user_table, movie_table, W1, b1, W2, b2):
    table = jnp.concatenate([user_table, movie_table], axis=1)
    u_rows, m_rows = _sc_gather(user, movie, table)
    y = _mlp(u_rows, m_rows, W1, b1, W2, b2)
    return y.reshape(_BATCH)


# speedup vs baseline: 1.4138x; 1.0150x over previous
import jax
import jax.numpy as jnp
from jax import lax
from jax.experimental import pallas as pl
from jax.experimental.pallas import tpu as pltpu
from jax.experimental.pallas import tpu_sc as plsc

_NC = 2
_NS = 16
_NW = _NC * _NS

_BATCH = 16384
_EMBED = 64
_ROW = 2 * _EMBED
_B_PER_W = _BATCH // _NW
_CHUNK = 256
_N_CHUNKS = _B_PER_W // _CHUNK


def _sc_gather_body(user_idx_hbm, movie_idx_hbm, table_hbm,
                    out_u_hbm, out_m_hbm,
                    uidx_v, midx_v, ubuf_v, mbuf_v, sem_u, sem_m):
    wid = lax.axis_index("s") * _NC + lax.axis_index("c")
    base = wid * _B_PER_W
    pltpu.sync_copy(user_idx_hbm.at[pl.ds(base, _B_PER_W)], uidx_v)
    pltpu.sync_copy(movie_idx_hbm.at[pl.ds(base, _B_PER_W)], midx_v)
    for ch in range(_N_CHUNKS):
        off = ch * _CHUNK
        cp_u = pltpu.async_copy(table_hbm.at[uidx_v.at[pl.ds(off, _CHUNK)]],
                                ubuf_v, sem_u)
        cp_m = pltpu.async_copy(table_hbm.at[midx_v.at[pl.ds(off, _CHUNK)]],
                                mbuf_v, sem_m)
        cp_u.wait()
        cp_m.wait()
        pltpu.sync_copy(ubuf_v, out_u_hbm.at[pl.ds(base + off, _CHUNK)])
        pltpu.sync_copy(mbuf_v, out_m_hbm.at[pl.ds(base + off, _CHUNK)])


def _sc_gather(user_idx, movie_idx, table):
    mesh = plsc.VectorSubcoreMesh(core_axis_name="c", subcore_axis_name="s",
                                  num_cores=_NC, num_subcores=_NS)
    return pl.kernel(
        _sc_gather_body,
        out_type=[
            jax.ShapeDtypeStruct((_BATCH, _ROW), jnp.float32),
            jax.ShapeDtypeStruct((_BATCH, _ROW), jnp.float32),
        ],
        mesh=mesh,
        scratch_types=[
            pltpu.VMEM((_B_PER_W,), jnp.int32),
            pltpu.VMEM((_B_PER_W,), jnp.int32),
            pltpu.VMEM((_CHUNK, _ROW), jnp.float32),
            pltpu.VMEM((_CHUNK, _ROW), jnp.float32),
            pltpu.SemaphoreType.DMA,
            pltpu.SemaphoreType.DMA,
        ],
    )(user_idx, movie_idx, table)


def _mlp_body(u_ref, m_ref, w1_ref, b1_ref, w2_ref, b2_ref, out_ref):
    u = u_ref[...]
    m = m_ref[...]
    w1 = w1_ref[...]
    dn = (((1,), (1,)), ((), ()))
    h = lax.dot_general(u[:, :_EMBED], w1[:, :_EMBED], dn,
                        preferred_element_type=jnp.float32)
    h += lax.dot_general(m[:, _EMBED:], w1[:, _EMBED:], dn,
                         preferred_element_type=jnp.float32)
    h = jnp.maximum(h + b1_ref[...], 0.0)
    y = jnp.sum(h * w2_ref[...], axis=1) + b2_ref[0, 0]
    out_ref[...] = y


def _mlp(u_rows, m_rows, W1, b1, W2, b2):
    BR = 2048
    grid = (_BATCH // BR,)
    return pl.pallas_call(
        _mlp_body,
        grid=grid,
        in_specs=[
            pl.BlockSpec((BR, _ROW), lambda i: (i, 0)),
            pl.BlockSpec((BR, _ROW), lambda i: (i, 0)),
            pl.BlockSpec((128, _ROW), lambda i: (0, 0)),
            pl.BlockSpec((1, 128), lambda i: (0, 0)),
            pl.BlockSpec((1, 128), lambda i: (0, 0)),
            pl.BlockSpec((1, 1), lambda i: (0, 0)),
        ],
        out_specs=pl.BlockSpec((BR,), lambda i: (i,)),
        out_shape=jax.ShapeDtypeStruct((_BATCH,), jnp.float32),
    )(u_rows, m_rows, W1, b1.reshape(1, 128), W2, b2.reshape(1, 1))


@jax.jit
def kernel(user, movie, user_table, movie_table, W1, b1, W2, b2):
    table = jnp.concatenate([user_table.T, movie_table.T], axis=0).T
    u_rows, m_rows = _sc_gather(user, movie, table)
    return _mlp(u_rows, m_rows, W1, b1, W2, b2)


# speedup vs baseline: 1.4160x; 1.0016x over previous
import jax
import jax.numpy as jnp
from jax import lax
from jax.experimental import pallas as pl
from jax.experimental.pallas import tpu as pltpu
from jax.experimental.pallas import tpu_sc as plsc

_NC = 2
_NS = 16
_NW = _NC * _NS

_BATCH = 16384
_EMBED = 64
_ROW = 2 * _EMBED
_B_PER_W = _BATCH // _NW
_CHUNK = 256
_N_CHUNKS = _B_PER_W // _CHUNK


def _sc_gather_body(user_idx_hbm, movie_idx_hbm, table_hbm,
                    out_u_hbm, out_m_hbm,
                    uidx_v, midx_v, ubuf_v, mbuf_v, sem_u, sem_m):
    wid = lax.axis_index("s") * _NC + lax.axis_index("c")
    base = wid * _B_PER_W
    pltpu.sync_copy(user_idx_hbm.at[pl.ds(base, _B_PER_W)], uidx_v)
    pltpu.sync_copy(movie_idx_hbm.at[pl.ds(base, _B_PER_W)], midx_v)
    for ch in range(_N_CHUNKS):
        off = ch * _CHUNK
        cp_u = pltpu.async_copy(table_hbm.at[uidx_v.at[pl.ds(off, _CHUNK)]],
                                ubuf_v, sem_u)
        cp_m = pltpu.async_copy(table_hbm.at[midx_v.at[pl.ds(off, _CHUNK)]],
                                mbuf_v, sem_m)
        cp_u.wait()
        cp_m.wait()
        pltpu.sync_copy(ubuf_v, out_u_hbm.at[pl.ds(base + off, _CHUNK)])
        pltpu.sync_copy(mbuf_v, out_m_hbm.at[pl.ds(base + off, _CHUNK)])


def _sc_gather(user_idx, movie_idx, table):
    mesh = plsc.VectorSubcoreMesh(core_axis_name="c", subcore_axis_name="s",
                                  num_cores=_NC, num_subcores=_NS)
    return pl.kernel(
        _sc_gather_body,
        out_type=[
            jax.ShapeDtypeStruct((_BATCH, _ROW), jnp.float32),
            jax.ShapeDtypeStruct((_BATCH, _ROW), jnp.float32),
        ],
        mesh=mesh,
        scratch_types=[
            pltpu.VMEM((_B_PER_W,), jnp.int32),
            pltpu.VMEM((_B_PER_W,), jnp.int32),
            pltpu.VMEM((_CHUNK, _ROW), jnp.float32),
            pltpu.VMEM((_CHUNK, _ROW), jnp.float32),
            pltpu.SemaphoreType.DMA,
            pltpu.SemaphoreType.DMA,
        ],
    )(user_idx, movie_idx, table)


def _mlp_body(u_ref, m_ref, w1_ref, b1_ref, w2_ref, b2_ref, out_ref):
    u = u_ref[...]
    m = m_ref[...]
    w1 = w1_ref[...]
    dn = (((1,), (1,)), ((), ()))
    h = lax.dot_general(u[:, :_EMBED], w1[:, :_EMBED], dn,
                        preferred_element_type=jnp.float32)
    h += lax.dot_general(m[:, _EMBED:], w1[:, _EMBED:], dn,
                         preferred_element_type=jnp.float32)
    h = jnp.maximum(h + b1_ref[...], 0.0)
    y = jnp.sum(h * w2_ref[...], axis=1) + b2_ref[0, 0]
    out_ref[...] = y


def _mlp(u_rows, m_rows, W1, b1, W2, b2):
    BR = 2048
    grid = (_BATCH // BR,)
    return pl.pallas_call(
        _mlp_body,
        grid=grid,
        in_specs=[
            pl.BlockSpec((BR, _ROW), lambda i: (i, 0)),
            pl.BlockSpec((BR, _ROW), lambda i: (i, 0)),
            pl.BlockSpec((128, _ROW), lambda i: (0, 0)),
            pl.BlockSpec((1, 128), lambda i: (0, 0)),
            pl.BlockSpec((1, 128), lambda i: (0, 0)),
            pl.BlockSpec((1, 1), lambda i: (0, 0)),
        ],
        out_specs=pl.BlockSpec((BR,), lambda i: (i,)),
        out_shape=jax.ShapeDtypeStruct((_BATCH,), jnp.float32),
    )(u_rows, m_rows, W1, b1.reshape(1, 128), W2, b2.reshape(1, 1))


@jax.jit
def kernel(user, movie, user_table, movie_table, W1, b1, W2, b2):
    table = (jnp.pad(user_table, ((0, 0), (0, _EMBED)))
             + jnp.pad(movie_table, ((0, 0), (_EMBED, 0))))
    u_rows, m_rows = _sc_gather(user, movie, table)
    return _mlp(u_rows, m_rows, W1, b1, W2, b2)


# speedup vs baseline: 1.4603x; 1.0313x over previous
import jax
import jax.numpy as jnp
from jax import lax
from jax.experimental import pallas as pl
from jax.experimental.pallas import tpu as pltpu
from jax.experimental.pallas import tpu_sc as plsc

_NC = 2
_NS = 16
_NW = _NC * _NS

_BATCH = 16384
_EMBED = 64
_ROW = 2 * _EMBED
_B_PER_W = _BATCH // _NW
_CHUNK = 256
_N_CHUNKS = _B_PER_W // _CHUNK
_LANES = 16


def _sc_gather_body(user_idx_hbm, movie_idx_hbm, table_hbm, x_hbm,
                    uidx_v, midx_v, xbuf_v, mbuf_v, sem_u, sem_m):
    wid = lax.axis_index("s") * _NC + lax.axis_index("c")
    base = wid * _B_PER_W
    pltpu.sync_copy(user_idx_hbm.at[pl.ds(base, _B_PER_W)], uidx_v)
    pltpu.sync_copy(movie_idx_hbm.at[pl.ds(base, _B_PER_W)], midx_v)
    for ch in range(_N_CHUNKS):
        off = ch * _CHUNK
        cp_u = pltpu.async_copy(table_hbm.at[uidx_v.at[pl.ds(off, _CHUNK)]],
                                xbuf_v, sem_u)
        cp_m = pltpu.async_copy(table_hbm.at[midx_v.at[pl.ds(off, _CHUNK)]],
                                mbuf_v, sem_m)
        cp_u.wait()
        cp_m.wait()

        def move_row(r, _):
            for c in range(_EMBED // _LANES):
                col = _EMBED + c * _LANES
                xbuf_v[r, pl.ds(col, _LANES)] = mbuf_v[r, pl.ds(col, _LANES)]
            return 0

        lax.fori_loop(0, _CHUNK, move_row, 0)
        pltpu.sync_copy(xbuf_v, x_hbm.at[pl.ds(base + off, _CHUNK)])


def _sc_gather(user_idx, movie_idx, table):
    mesh = plsc.VectorSubcoreMesh(core_axis_name="c", subcore_axis_name="s",
                                  num_cores=_NC, num_subcores=_NS)
    return pl.kernel(
        _sc_gather_body,
        out_type=jax.ShapeDtypeStruct((_BATCH, _ROW), jnp.float32),
        mesh=mesh,
        scratch_types=[
            pltpu.VMEM((_B_PER_W,), jnp.int32),
            pltpu.VMEM((_B_PER_W,), jnp.int32),
            pltpu.VMEM((_CHUNK, _ROW), jnp.float32),
            pltpu.VMEM((_CHUNK, _ROW), jnp.float32),
            pltpu.SemaphoreType.DMA,
            pltpu.SemaphoreType.DMA,
        ],
    )(user_idx, movie_idx, table)


def _mlp_body(x_ref, w1_ref, b1_ref, w2_ref, b2_ref, out_ref):
    x = x_ref[...]
    w1 = w1_ref[...]
    dn = (((1,), (1,)), ((), ()))
    h = lax.dot_general(x, w1, dn, preferred_element_type=jnp.float32)
    h = jnp.maximum(h + b1_ref[...], 0.0)
    y = jnp.sum(h * w2_ref[...], axis=1) + b2_ref[0, 0]
    out_ref[...] = y


def _mlp(x, W1, b1, W2, b2):
    BR = 2048
    grid = (_BATCH // BR,)
    return pl.pallas_call(
        _mlp_body,
        grid=grid,
        in_specs=[
            pl.BlockSpec((BR, _ROW), lambda i: (i, 0)),
            pl.BlockSpec((128, _ROW), lambda i: (0, 0)),
            pl.BlockSpec((1, 128), lambda i: (0, 0)),
            pl.BlockSpec((1, 128), lambda i: (0, 0)),
            pl.BlockSpec((1, 1), lambda i: (0, 0)),
        ],
        out_specs=pl.BlockSpec((BR,), lambda i: (i,)),
        out_shape=jax.ShapeDtypeStruct((_BATCH,), jnp.float32),
    )(x, W1, b1.reshape(1, 128), W2, b2.reshape(1, 1))


@jax.jit
def kernel(user, movie, user_table, movie_table, W1, b1, W2, b2):
    table = jnp.concatenate([user_table, movie_table], axis=1)
    x = _sc_gather(user, movie, table)
    return _mlp(x, W1, b1, W2, b2)
